# baseline (speedup 1.0000x reference)
import functools

import jax
import jax.numpy as jnp
from jax import lax
from jax.experimental import pallas as pl
from jax.experimental.pallas import tpu as pltpu
from jax.experimental.pallas import tpu_sc as plsc

NUM_USERS = 1000000
NUM_ITEMS = 1000000
EMBED_DIM = 64
BATCH = 16384

_NC = 2
_NS = 16
_NW = _NC * _NS
_BPW = BATCH // _NW
_CHUNK = 128
_NCHUNK = _BPW // _CHUNK
_GROUP = 16
_NGROUP = _BPW // _GROUP
_L = 16
_NV = EMBED_DIM // _L


def _sc_body(users_hbm, pos_hbm, neg_hbm, utab_hbm, itab_hbm,
             pos_out, neg_out, reg_out,
             uidx_v, pidx_v, nidx_v, urows_v, prows_v, nrows_v,
             csp_v, csn_v, psc_v, nsc_v, acc_v, sem):
    wid = lax.axis_index("s") * _NC + lax.axis_index("c")

    pltpu.sync_copy(users_hbm.at[wid], uidx_v)
    pltpu.sync_copy(pos_hbm.at[wid], pidx_v)
    pltpu.sync_copy(neg_hbm.at[wid], nidx_v)

    copies = []
    for j in range(_NCHUNK):
        sl = pl.ds(j * _CHUNK, _CHUNK)
        copies.append(pltpu.make_async_copy(
            utab_hbm.at[uidx_v.at[j]], urows_v.at[sl], sem))
        copies.append(pltpu.make_async_copy(
            itab_hbm.at[pidx_v.at[j]], prows_v.at[sl], sem))
        copies.append(pltpu.make_async_copy(
            itab_hbm.at[nidx_v.at[j]], nrows_v.at[sl], sem))
    for c in copies:
        c.start()
    for c in copies:
        c.wait()

    lane15 = lax.iota(jnp.int32, _L) * _L + (_L - 1)

    def group_body(g, acc):
        for i in range(_GROUP):
            b = g * _GROUP + i
            dot_p = None
            dot_n = None
            for c in range(_NV):
                sl = pl.ds(c * _L, _L)
                u = urows_v[b, sl]
                p = prows_v[b, sl]
                n = nrows_v[b, sl]
                acc = acc + u * u + p * p + n * n
                if dot_p is None:
                    dot_p = u * p
                    dot_n = u * n
                else:
                    dot_p = dot_p + u * p
                    dot_n = dot_n + u * n
            csp_v[pl.ds(i * _L, _L)] = plsc.cumsum(dot_p)
            csn_v[pl.ds(i * _L, _L)] = plsc.cumsum(dot_n)
        psc_v[pl.ds(g * _GROUP, _GROUP)] = plsc.load_gather(csp_v, [lane15])
        nsc_v[pl.ds(g * _GROUP, _GROUP)] = plsc.load_gather(csn_v, [lane15])
        return acc

    acc = lax.fori_loop(0, _NGROUP, group_body,
                        jnp.zeros((_L,), jnp.float32), unroll=False)
    acc_v[...] = acc

    base = wid * _BPW
    pltpu.sync_copy(psc_v, pos_out.at[pl.ds(base, _BPW)])
    pltpu.sync_copy(nsc_v, neg_out.at[pl.ds(base, _BPW)])
    pltpu.sync_copy(acc_v, reg_out.at[wid])


@jax.jit
def _bprmf_sc(users3, pos3, neg3, user_table, item_table):
    mesh = plsc.VectorSubcoreMesh(core_axis_name="c", subcore_axis_name="s")
    f = functools.partial(
        pl.kernel,
        mesh=mesh,
        compiler_params=pltpu.CompilerParams(
            needs_layout_passes=False, use_tc_tiling_on_sc=False),
        out_type=(
            jax.ShapeDtypeStruct((BATCH,), jnp.float32),
            jax.ShapeDtypeStruct((BATCH,), jnp.float32),
            jax.ShapeDtypeStruct((_NW, _L), jnp.float32),
        ),
        scratch_types=[
            pltpu.VMEM((_NCHUNK, _CHUNK), jnp.int32),
            pltpu.VMEM((_NCHUNK, _CHUNK), jnp.int32),
            pltpu.VMEM((_NCHUNK, _CHUNK), jnp.int32),
            pltpu.VMEM((_BPW, EMBED_DIM), jnp.float32),
            pltpu.VMEM((_BPW, EMBED_DIM), jnp.float32),
            pltpu.VMEM((_BPW, EMBED_DIM), jnp.float32),
            pltpu.VMEM((_GROUP * _L,), jnp.float32),
            pltpu.VMEM((_GROUP * _L,), jnp.float32),
            pltpu.VMEM((_BPW,), jnp.float32),
            pltpu.VMEM((_BPW,), jnp.float32),
            pltpu.VMEM((_L,), jnp.float32),
            pltpu.SemaphoreType.DMA,
        ],
    )(_sc_body)
    return f(users3, pos3, neg3, user_table, item_table)


def kernel(users, pos_items, neg_items, user_table, item_table):
    users3 = users.reshape(_NW, _NCHUNK, _CHUNK)
    pos3 = pos_items.reshape(_NW, _NCHUNK, _CHUNK)
    neg3 = neg_items.reshape(_NW, _NCHUNK, _CHUNK)
    pos_scores, neg_scores, reg_part = _bprmf_sc(
        users3, pos3, neg3, user_table, item_table)
    reg_loss = 0.5 * jnp.sum(reg_part) / float(BATCH)
    return (pos_scores, neg_scores, reg_loss)
